# baseline (speedup 1.0000x reference)
import functools

import jax
import jax.numpy as jnp
from jax import lax
from jax.experimental import pallas as pl
from jax.experimental.pallas import tpu as pltpu
from jax.experimental.pallas import tpu_sc as plsc

_N = 50000
_D = 256
_C = 1024

_NC = 2
_NS = 16
_NW = _NC * _NS
_CPW = 13
_Q = _CPW * 128
_NP = _NW * _Q
_A = 1024

_BN = 2000
_NB = _N // _BN


def _proj_pool_body(ids_ref, x_ref, wg_ref, bg_ref, gg_ref, gb_ref,
                    wf_ref, bf_ref, fg_ref, fb_ref, o_ref,
                    acc_ref, cnt_ref):
    i = pl.program_id(0)
    x = x_ref[...].astype(jnp.bfloat16)

    def ln(h, gamma, beta):
        mu = jnp.mean(h, axis=1, keepdims=True)
        ms = jnp.mean(h * h, axis=1, keepdims=True)
        var = ms - mu * mu
        return (h - mu) * lax.rsqrt(var + 1e-5) * gamma + beta

    hg = jnp.dot(x, wg_ref[...], preferred_element_type=jnp.float32) + bg_ref[...]
    gates = jax.nn.sigmoid(ln(hg, gg_ref[...], gb_ref[...]))

    hf = jnp.dot(x, wf_ref[...], preferred_element_type=jnp.float32) + bf_ref[...]
    hf = ln(hf, fg_ref[...], fb_ref[...])
    feats = 0.5 * hf * (1.0 + lax.erf(hf * 0.7071067811865476))

    gated = gates * feats

    ids = ids_ref[0]
    clusters = lax.broadcasted_iota(jnp.int32, (_A, _BN), 0)
    oh_t = (clusters == ids).astype(jnp.bfloat16)
    sums_part = jax.lax.dot_general(
        oh_t, gated.astype(jnp.bfloat16),
        dimension_numbers=(((1,), (0,)), ((), ())),
        preferred_element_type=jnp.float32)
    cnt_part = jax.lax.dot_general(
        oh_t, jnp.ones((_BN, 8), jnp.bfloat16),
        dimension_numbers=(((1,), (0,)), ((), ())),
        preferred_element_type=jnp.float32)

    @pl.when(i == 0)
    def _init():
        acc_ref[...] = jnp.zeros_like(acc_ref)
        cnt_ref[...] = jnp.zeros_like(cnt_ref)

    acc_ref[...] += sums_part
    cnt_ref[...] += cnt_part

    @pl.when(i == _NB - 1)
    def _finish():
        cnt = jnp.maximum(cnt_ref[:, 0], 1.0)
        o_ref[...] = acc_ref[...] / cnt[:, None]


def _proj_pool(ids3, x, wgt, bg, gg, gb, wft, bf, fg, fb):
    row_spec = pl.BlockSpec((_BN, _D), lambda i: (i, 0))
    mat_spec = pl.BlockSpec((_D, _D), lambda i: (0, 0))
    vec_spec = pl.BlockSpec((1, _D), lambda i: (0, 0))
    ids_spec = pl.BlockSpec((1, 1, _BN), lambda i: (i, 0, 0))
    return pl.pallas_call(
        _proj_pool_body,
        grid=(_NB,),
        in_specs=[ids_spec, row_spec, mat_spec, vec_spec, vec_spec, vec_spec,
                  mat_spec, vec_spec, vec_spec, vec_spec],
        out_specs=pl.BlockSpec((_A, _D), lambda i: (0, 0)),
        out_shape=jax.ShapeDtypeStruct((_A, _D), jnp.float32),
        scratch_shapes=[
            pltpu.VMEM((_A, _D), jnp.float32),
            pltpu.VMEM((_A, 8), jnp.float32),
        ],
    )(ids3, x, wgt, bg, gg, gb, wft, bf, fg, fb)


_RSC = 24960
_RTC = _N - _RSC
_BG = 2504
_NBG = _RTC // _BG


def _tc_gather_body(ids_ref, pooled_ref, o_ref):
    ids = ids_ref[0]
    clusters = lax.broadcasted_iota(jnp.int32, (_A, _BG), 0)
    oh_t = (clusters == ids).astype(jnp.bfloat16)
    o_ref[...] = jax.lax.dot_general(
        oh_t, pooled_ref[...].astype(jnp.bfloat16),
        dimension_numbers=(((0,), (0,)), ((), ())),
        preferred_element_type=jnp.float32)


def _tc_gather(ids3g, pooled):
    return pl.pallas_call(
        _tc_gather_body,
        grid=(_NBG,),
        in_specs=[pl.BlockSpec((1, 1, _BG), lambda i: (i, 0, 0)),
                  pl.BlockSpec((_A, _D), lambda i: (0, 0))],
        out_specs=pl.BlockSpec((_BG, _D), lambda i: (i, 0)),
        out_shape=jax.ShapeDtypeStruct((_RTC, _D), jnp.float32),
    )(ids3g, pooled)


_MESH = plsc.VectorSubcoreMesh(core_axis_name="c", subcore_axis_name="s",
                               num_cores=_NC, num_subcores=_NS)


@functools.partial(
    pl.kernel,
    out_type=jax.ShapeDtypeStruct((_RSC, _D), jnp.float32),
    mesh=_MESH,
    scratch_types=[
        pltpu.VMEM((_CPW, 128), jnp.int32),
        pltpu.VMEM((_CPW, 128), jnp.int32),
        pltpu.VMEM((2, 128, 128), jnp.float32),
        pltpu.VMEM((2, 128, 128), jnp.float32),
        pltpu.SemaphoreType.DMA,
        pltpu.SemaphoreType.DMA,
        pltpu.SemaphoreType.DMA,
        pltpu.SemaphoreType.DMA,
    ],
)
def _sc_gather(pooled2_hbm, ca3lo_hbm, ca3hi_hbm, out_hbm,
               ilo_v, ihi_v, blo_v, bhi_v, lsem0, lsem1, hsem0, hsem1):
    c = lax.axis_index("c")
    s = lax.axis_index("s")
    w = s * _NC + c

    pltpu.sync_copy(ca3lo_hbm.at[w], ilo_v)
    pltpu.sync_copy(ca3hi_hbm.at[w], ihi_v)
    base = w * _Q
    lsems = (lsem0, lsem1)
    hsems = (hsem0, hsem1)

    def start(j, b):
        @pl.when(base + j * 128 < _RSC)
        def _():
            pltpu.async_copy(pooled2_hbm.at[ilo_v.at[j]], blo_v.at[b], lsems[b])
            pltpu.async_copy(pooled2_hbm.at[ihi_v.at[j]], bhi_v.at[b], hsems[b])

    start(0, 0)
    for j in range(_CPW):
        if j + 1 < _CPW:
            start(j + 1, (j + 1) % 2)
        b = j % 2
        lo = base + j * 128

        @pl.when(lo < _RSC)
        def _wait():
            pltpu.make_async_copy(pooled2_hbm.at[ilo_v.at[j]],
                                  blo_v.at[b], lsems[b]).wait()
            pltpu.make_async_copy(pooled2_hbm.at[ihi_v.at[j]],
                                  bhi_v.at[b], hsems[b]).wait()

        @pl.when(lo + 128 <= _RSC)
        def _store_full():
            pltpu.sync_copy(blo_v.at[b],
                            out_hbm.at[pl.ds(lo, 128), pl.ds(0, 128)])
            pltpu.sync_copy(bhi_v.at[b],
                            out_hbm.at[pl.ds(lo, 128), pl.ds(128, 128)])


def kernel(x, cluster_assignments, batch, Wg, bg, g_gamma, g_beta,
           Wf, bf, f_gamma, f_beta):
    del batch

    ids3 = cluster_assignments.reshape(_NB, 1, _BN)
    ca_sc = jnp.zeros((_NP,), jnp.int32).at[:_RSC].set(cluster_assignments[:_RSC])
    ca3lo = (ca_sc * 2).reshape(_NW, _CPW, 128)
    ca3hi = (ca_sc * 2 + 1).reshape(_NW, _CPW, 128)
    ids3g = cluster_assignments[_RSC:].reshape(_NBG, 1, _BG)

    pooled = _proj_pool(ids3, x, Wg.T.astype(jnp.bfloat16), bg.reshape(1, _D),
                        g_gamma.reshape(1, _D), g_beta.reshape(1, _D),
                        Wf.T.astype(jnp.bfloat16), bf.reshape(1, _D),
                        f_gamma.reshape(1, _D), f_beta.reshape(1, _D))

    pooled2 = pooled.reshape(2 * _A, 128)
    out_sc = _sc_gather(pooled2, ca3lo, ca3hi)
    out_tc = _tc_gather(ids3g, pooled)
    return jnp.concatenate([out_sc, out_tc], axis=0)


# speedup vs baseline: 1.0971x; 1.0971x over previous
import functools

import jax
import jax.numpy as jnp
from jax import lax
from jax.experimental import pallas as pl
from jax.experimental.pallas import tpu as pltpu
from jax.experimental.pallas import tpu_sc as plsc

_N = 50000
_D = 256
_C = 1024

_NC = 2
_NS = 16
_NW = _NC * _NS
_CPW = 13
_Q = _CPW * 128
_NP = _NW * _Q
_A = 1024

_BN = 2000
_NB = _N // _BN


def _proj_pool_body(ids_ref, x_ref, wg_ref, bg_ref, gg_ref, gb_ref,
                    wf_ref, bf_ref, fg_ref, fb_ref, o_ref,
                    acc_ref, cnt_ref):
    i = pl.program_id(0)
    x = x_ref[...].astype(jnp.bfloat16)

    def ln(h, gamma, beta):
        mu = jnp.mean(h, axis=1, keepdims=True)
        ms = jnp.mean(h * h, axis=1, keepdims=True)
        var = ms - mu * mu
        return (h - mu) * lax.rsqrt(var + 1e-5) * gamma + beta

    hg = jnp.dot(x, wg_ref[...], preferred_element_type=jnp.float32) + bg_ref[...]
    gates = jax.nn.sigmoid(ln(hg, gg_ref[...], gb_ref[...]))

    hf = jnp.dot(x, wf_ref[...], preferred_element_type=jnp.float32) + bf_ref[...]
    hf = ln(hf, fg_ref[...], fb_ref[...])
    feats = 0.5 * hf * (1.0 + lax.erf(hf * 0.7071067811865476))

    gated = gates * feats

    ids = ids_ref[0]
    clusters = lax.broadcasted_iota(jnp.int32, (_A, _BN), 0)
    oh_t = (clusters == ids).astype(jnp.bfloat16)
    sums_part = jax.lax.dot_general(
        oh_t, gated.astype(jnp.bfloat16),
        dimension_numbers=(((1,), (0,)), ((), ())),
        preferred_element_type=jnp.float32)
    cnt_part = jax.lax.dot_general(
        oh_t, jnp.ones((_BN, 8), jnp.bfloat16),
        dimension_numbers=(((1,), (0,)), ((), ())),
        preferred_element_type=jnp.float32)

    @pl.when(i == 0)
    def _init():
        acc_ref[...] = jnp.zeros_like(acc_ref)
        cnt_ref[...] = jnp.zeros_like(cnt_ref)

    acc_ref[...] += sums_part
    cnt_ref[...] += cnt_part

    @pl.when(i == _NB - 1)
    def _finish():
        cnt = jnp.maximum(cnt_ref[:, 0], 1.0)
        o_ref[...] = acc_ref[...] / cnt[:, None]


def _proj_pool(ids3, x, wgt, bg, gg, gb, wft, bf, fg, fb):
    row_spec = pl.BlockSpec((_BN, _D), lambda i: (i, 0))
    mat_spec = pl.BlockSpec((_D, _D), lambda i: (0, 0))
    vec_spec = pl.BlockSpec((1, _D), lambda i: (0, 0))
    ids_spec = pl.BlockSpec((1, 1, _BN), lambda i: (i, 0, 0))
    return pl.pallas_call(
        _proj_pool_body,
        grid=(_NB,),
        in_specs=[ids_spec, row_spec, mat_spec, vec_spec, vec_spec, vec_spec,
                  mat_spec, vec_spec, vec_spec, vec_spec],
        out_specs=pl.BlockSpec((_A, _D), lambda i: (0, 0)),
        out_shape=jax.ShapeDtypeStruct((_A, _D), jnp.float32),
        scratch_shapes=[
            pltpu.VMEM((_A, _D), jnp.float32),
            pltpu.VMEM((_A, 8), jnp.float32),
        ],
    )(ids3, x, wgt, bg, gg, gb, wft, bf, fg, fb)


_MESH = plsc.VectorSubcoreMesh(core_axis_name="c", subcore_axis_name="s",
                               num_cores=_NC, num_subcores=_NS)


@functools.partial(
    pl.kernel,
    out_type=jax.ShapeDtypeStruct((_N, _D), jnp.float32),
    mesh=_MESH,
    scratch_types=[
        pltpu.VMEM((_CPW, 128), jnp.int32),
        pltpu.VMEM((_CPW, 128), jnp.int32),
        pltpu.VMEM((2, 128, 128), jnp.float32),
        pltpu.VMEM((2, 128, 128), jnp.float32),
        pltpu.SemaphoreType.DMA,
        pltpu.SemaphoreType.DMA,
        pltpu.SemaphoreType.DMA,
        pltpu.SemaphoreType.DMA,
    ],
)
def _sc_gather(pooled2_hbm, ca3lo_hbm, ca3hi_hbm, out_hbm,
               ilo_v, ihi_v, blo_v, bhi_v, lsem0, lsem1, hsem0, hsem1):
    c = lax.axis_index("c")
    s = lax.axis_index("s")
    w = s * _NC + c

    pltpu.sync_copy(ca3lo_hbm.at[w], ilo_v)
    pltpu.sync_copy(ca3hi_hbm.at[w], ihi_v)
    base = w * _Q
    lsems = (lsem0, lsem1)
    hsems = (hsem0, hsem1)

    def start(j, b):
        @pl.when(base + j * 128 < _N)
        def _():
            pltpu.async_copy(pooled2_hbm.at[ilo_v.at[j]], blo_v.at[b], lsems[b])
            pltpu.async_copy(pooled2_hbm.at[ihi_v.at[j]], bhi_v.at[b], hsems[b])

    start(0, 0)
    for j in range(_CPW):
        if j + 1 < _CPW:
            start(j + 1, (j + 1) % 2)
        b = j % 2
        lo = base + j * 128

        @pl.when(lo < _N)
        def _wait():
            pltpu.make_async_copy(pooled2_hbm.at[ilo_v.at[j]],
                                  blo_v.at[b], lsems[b]).wait()
            pltpu.make_async_copy(pooled2_hbm.at[ihi_v.at[j]],
                                  bhi_v.at[b], hsems[b]).wait()

        @pl.when(lo + 128 <= _N)
        def _store_full():
            pltpu.sync_copy(blo_v.at[b],
                            out_hbm.at[pl.ds(lo, 128), pl.ds(0, 128)])
            pltpu.sync_copy(bhi_v.at[b],
                            out_hbm.at[pl.ds(lo, 128), pl.ds(128, 128)])

        @pl.when((lo < _N) & (lo + 128 > _N))
        def _store_tail():
            tail = _N % 128
            pltpu.sync_copy(blo_v.at[b].at[pl.ds(0, tail)],
                            out_hbm.at[pl.ds(_N - tail, tail), pl.ds(0, 128)])
            pltpu.sync_copy(bhi_v.at[b].at[pl.ds(0, tail)],
                            out_hbm.at[pl.ds(_N - tail, tail), pl.ds(128, 128)])


def kernel(x, cluster_assignments, batch, Wg, bg, g_gamma, g_beta,
           Wf, bf, f_gamma, f_beta):
    del batch

    ids3 = cluster_assignments.reshape(_NB, 1, _BN)
    ca_p = jnp.zeros((_NP,), jnp.int32).at[:_N].set(cluster_assignments)
    ca3lo = (ca_p * 2).reshape(_NW, _CPW, 128)
    ca3hi = (ca_p * 2 + 1).reshape(_NW, _CPW, 128)

    pooled = _proj_pool(ids3, x, Wg.T.astype(jnp.bfloat16), bg.reshape(1, _D),
                        g_gamma.reshape(1, _D), g_beta.reshape(1, _D),
                        Wf.T.astype(jnp.bfloat16), bf.reshape(1, _D),
                        f_gamma.reshape(1, _D), f_beta.reshape(1, _D))

    pooled2 = pooled.reshape(2 * _A, 128)
    return _sc_gather(pooled2, ca3lo, ca3hi)


# speedup vs baseline: 1.0988x; 1.0016x over previous
import functools

import jax
import jax.numpy as jnp
from jax import lax
from jax.experimental import pallas as pl
from jax.experimental.pallas import tpu as pltpu
from jax.experimental.pallas import tpu_sc as plsc

_N = 50000
_D = 256
_C = 1024

_NC = 2
_NS = 16
_NW = _NC * _NS
_CPW = 13
_Q = _CPW * 128
_NP = _NW * _Q
_A = 1024

_BN = 2000
_NB = _N // _BN


def _proj_pool_body(ids_ref, x_ref, wg_ref, bg_ref, gg_ref, gb_ref,
                    wf_ref, bf_ref, fg_ref, fb_ref, o_ref,
                    acc_ref, cnt_ref):
    i = pl.program_id(0)
    x = x_ref[...].astype(jnp.bfloat16)

    def ln(h, gamma, beta):
        mu = jnp.mean(h, axis=1, keepdims=True)
        ms = jnp.mean(h * h, axis=1, keepdims=True)
        var = ms - mu * mu
        return (h - mu) * lax.rsqrt(var + 1e-5) * gamma + beta

    hg = jnp.dot(x, wg_ref[...], preferred_element_type=jnp.float32) + bg_ref[...]
    gates = jax.nn.sigmoid(ln(hg, gg_ref[...], gb_ref[...]))

    hf = jnp.dot(x, wf_ref[...], preferred_element_type=jnp.float32) + bf_ref[...]
    hf = ln(hf, fg_ref[...], fb_ref[...])
    feats = 0.5 * hf * (1.0 + lax.erf(hf * 0.7071067811865476))

    gated = gates * feats

    ids = ids_ref[0]
    clusters = lax.broadcasted_iota(jnp.int32, (_A, _BN), 0)
    oh_t = (clusters == ids).astype(jnp.bfloat16)
    sums_part = jax.lax.dot_general(
        oh_t, gated.astype(jnp.bfloat16),
        dimension_numbers=(((1,), (0,)), ((), ())),
        preferred_element_type=jnp.float32)
    cnt_part = jax.lax.dot_general(
        oh_t, jnp.ones((_BN, 8), jnp.bfloat16),
        dimension_numbers=(((1,), (0,)), ((), ())),
        preferred_element_type=jnp.float32)

    @pl.when(i == 0)
    def _init():
        acc_ref[...] = jnp.zeros_like(acc_ref)
        cnt_ref[...] = jnp.zeros_like(cnt_ref)

    acc_ref[...] += sums_part
    cnt_ref[...] += cnt_part

    @pl.when(i == _NB - 1)
    def _finish():
        cnt = jnp.maximum(cnt_ref[:, 0], 1.0)
        o_ref[...] = acc_ref[...] / cnt[:, None]


def _proj_pool(ids3, x, wgt, bg, gg, gb, wft, bf, fg, fb):
    row_spec = pl.BlockSpec((_BN, _D), lambda i: (i, 0))
    mat_spec = pl.BlockSpec((_D, _D), lambda i: (0, 0))
    vec_spec = pl.BlockSpec((1, _D), lambda i: (0, 0))
    ids_spec = pl.BlockSpec((1, 1, _BN), lambda i: (i, 0, 0))
    return pl.pallas_call(
        _proj_pool_body,
        grid=(_NB,),
        in_specs=[ids_spec, row_spec, mat_spec, vec_spec, vec_spec, vec_spec,
                  mat_spec, vec_spec, vec_spec, vec_spec],
        out_specs=pl.BlockSpec((_A, _D), lambda i: (0, 0)),
        out_shape=jax.ShapeDtypeStruct((_A, _D), jnp.float32),
        scratch_shapes=[
            pltpu.VMEM((_A, _D), jnp.float32),
            pltpu.VMEM((_A, 8), jnp.float32),
        ],
    )(ids3, x, wgt, bg, gg, gb, wft, bf, fg, fb)


_MESH = plsc.VectorSubcoreMesh(core_axis_name="c", subcore_axis_name="s",
                               num_cores=_NC, num_subcores=_NS)


@functools.partial(
    pl.kernel,
    out_type=jax.ShapeDtypeStruct((_N, _D), jnp.float32),
    mesh=_MESH,
    scratch_types=[
        pltpu.VMEM((_CPW, 128), jnp.int32),
        pltpu.VMEM((_CPW, 128), jnp.int32),
        pltpu.VMEM((2, 128, 128), jnp.float32),
        pltpu.VMEM((2, 128, 128), jnp.float32),
        pltpu.SemaphoreType.DMA,
        pltpu.SemaphoreType.DMA,
        pltpu.SemaphoreType.DMA,
        pltpu.SemaphoreType.DMA,
        pltpu.SemaphoreType.DMA,
        pltpu.SemaphoreType.DMA,
        pltpu.SemaphoreType.DMA,
        pltpu.SemaphoreType.DMA,
    ],
)
def _sc_gather(pooled2_hbm, ca3lo_hbm, ca3hi_hbm, out_hbm,
               ilo_v, ihi_v, blo_v, bhi_v,
               lsem0, lsem1, hsem0, hsem1, slsem0, slsem1, shsem0, shsem1):
    c = lax.axis_index("c")
    s = lax.axis_index("s")
    w = s * _NC + c

    pltpu.sync_copy(ca3lo_hbm.at[w], ilo_v)
    pltpu.sync_copy(ca3hi_hbm.at[w], ihi_v)
    base = w * _Q
    lsems = (lsem0, lsem1)
    hsems = (hsem0, hsem1)
    slsems = (slsem0, slsem1)
    shsems = (shsem0, shsem1)
    tail = _N % 128

    def full_dsc(j, b):
        lo = base + j * 128
        return (pltpu.make_async_copy(blo_v.at[b],
                                      out_hbm.at[pl.ds(lo, 128), pl.ds(0, 128)],
                                      slsems[b]),
                pltpu.make_async_copy(bhi_v.at[b],
                                      out_hbm.at[pl.ds(lo, 128), pl.ds(128, 128)],
                                      shsems[b]))

    def tail_dsc(b):
        return (pltpu.make_async_copy(blo_v.at[b].at[pl.ds(0, tail)],
                                      out_hbm.at[pl.ds(_N - tail, tail), pl.ds(0, 128)],
                                      slsems[b]),
                pltpu.make_async_copy(bhi_v.at[b].at[pl.ds(0, tail)],
                                      out_hbm.at[pl.ds(_N - tail, tail), pl.ds(128, 128)],
                                      shsems[b]))

    def start(j, b):
        @pl.when(base + j * 128 < _N)
        def _():
            pltpu.async_copy(pooled2_hbm.at[ilo_v.at[j]], blo_v.at[b], lsems[b])
            pltpu.async_copy(pooled2_hbm.at[ihi_v.at[j]], bhi_v.at[b], hsems[b])

    def wait_store(j, b):
        lo = base + j * 128

        @pl.when(lo + 128 <= _N)
        def _ws_full():
            d0, d1 = full_dsc(j, b)
            d0.wait()
            d1.wait()

        @pl.when((lo < _N) & (lo + 128 > _N))
        def _ws_tail():
            d0, d1 = tail_dsc(b)
            d0.wait()
            d1.wait()

    start(0, 0)
    for j in range(_CPW):
        b = j % 2
        lo = base + j * 128
        if j + 1 < _CPW:
            if j >= 1:
                wait_store(j - 1, (j + 1) % 2)
            start(j + 1, (j + 1) % 2)

        @pl.when(lo < _N)
        def _wait_gather():
            pltpu.make_async_copy(pooled2_hbm.at[ilo_v.at[j]],
                                  blo_v.at[b], lsems[b]).wait()
            pltpu.make_async_copy(pooled2_hbm.at[ihi_v.at[j]],
                                  bhi_v.at[b], hsems[b]).wait()

        @pl.when(lo + 128 <= _N)
        def _store_full():
            d0, d1 = full_dsc(j, b)
            d0.start()
            d1.start()

        @pl.when((lo < _N) & (lo + 128 > _N))
        def _store_tail():
            d0, d1 = tail_dsc(b)
            d0.start()
            d1.start()

    wait_store(_CPW - 2, (_CPW - 2) % 2)
    wait_store(_CPW - 1, (_CPW - 1) % 2)


def kernel(x, cluster_assignments, batch, Wg, bg, g_gamma, g_beta,
           Wf, bf, f_gamma, f_beta):
    del batch

    ids3 = cluster_assignments.reshape(_NB, 1, _BN)
    ca_p = jnp.zeros((_NP,), jnp.int32).at[:_N].set(cluster_assignments)
    ca3lo = (ca_p * 2).reshape(_NW, _CPW, 128)
    ca3hi = (ca_p * 2 + 1).reshape(_NW, _CPW, 128)

    pooled = _proj_pool(ids3, x, Wg.T.astype(jnp.bfloat16), bg.reshape(1, _D),
                        g_gamma.reshape(1, _D), g_beta.reshape(1, _D),
                        Wf.T.astype(jnp.bfloat16), bf.reshape(1, _D),
                        f_gamma.reshape(1, _D), f_beta.reshape(1, _D))

    pooled2 = pooled.reshape(2 * _A, 128)
    return _sc_gather(pooled2, ca3lo, ca3hi)
